# Initial kernel scaffold; baseline (speedup 1.0000x reference)
import jax
import jax.numpy as jnp
from jax.experimental import pallas as pl


def kernel(relation_embeddings, query_rels, query_entities, edge_index, edge_type, num_nodes, num_relations, W1, b1, W2, b2, Wg1, bg1, Wg2, bg2):
    raise NotImplementedError("write your pallas kernel here")


# speedup vs baseline: 12.3960x; 12.3960x over previous
import functools

import jax
import jax.numpy as jnp
from jax import lax
from jax.experimental import pallas as pl
from jax.experimental.pallas import tpu as pltpu
from jax.experimental.pallas import tpu_sc as plsc

N_NODES = 10000
NC, NS, L = 2, 16, 16

ROW = 128
HSTART = N_NODES * ROW
TBL = HSTART + ROW
ZSLICE = TBL // NS
ZBUF = 8192


def _sc_kernel(B, E):
    EP = E // (NC * NS)
    EPP = ((EP + 127) // 128) * 128
    NG = EPP // 128
    QT = B // NS

    mesh = plsc.VectorSubcoreMesh(core_axis_name="c", subcore_axis_name="s",
                                  num_cores=NC, num_subcores=NS)

    @functools.partial(
        pl.kernel,
        out_type=(
            jax.ShapeDtypeStruct((NC, B, ROW), jnp.float32),
            jax.ShapeDtypeStruct((NC, ROW), jnp.float32),
        ),
        mesh=mesh,
        scratch_types=[
            pltpu.VMEM_SHARED((TBL,), jnp.float32),
            pltpu.VMEM((EPP,), jnp.int32),
            pltpu.VMEM((EPP,), jnp.int32),
            pltpu.VMEM((EPP,), jnp.int32),
            pltpu.VMEM((3, 128), jnp.int32),
            pltpu.VMEM((3, 128), jnp.float32),
            pltpu.VMEM((L,), jnp.int32),
            pltpu.VMEM((QT, ROW), jnp.int32),
            pltpu.VMEM((QT, ROW), jnp.float32),
            pltpu.VMEM((ROW,), jnp.float32),
            pltpu.VMEM((ZBUF,), jnp.float32),
        ],
    )
    def sc_fn(src_hbm, dst_hbm, typ_hbm, qent_hbm,
              counts_out, hist_out,
              table, src_v, dst_v, typ_v, idx_b, val_b, q_v, idx_g, gbuf,
              hbuf, zbuf):
        c = lax.axis_index("c")
        s = lax.axis_index("s")
        wid = c * NS + s
        lane = jnp.arange(L, dtype=jnp.int32)

        zvec = jnp.zeros((L,), dtype=jnp.float32)

        def zfill(j, carry):
            zbuf[pl.ds(j * L, L)] = zvec
            return carry

        lax.fori_loop(0, ZBUF // L, zfill, 0)
        off = 0
        while off < ZSLICE:
            n = min(ZBUF, ZSLICE - off)
            pltpu.sync_copy(zbuf.at[pl.ds(0, n)],
                            table.at[pl.ds(s * ZSLICE + off, n)])
            off += n
        plsc.subcore_barrier()

        base = wid * EP
        pltpu.sync_copy(src_hbm.at[pl.ds(base, EP)], src_v.at[pl.ds(0, EP)])
        pltpu.sync_copy(dst_hbm.at[pl.ds(base, EP)], dst_v.at[pl.ds(0, EP)])
        pltpu.sync_copy(typ_hbm.at[pl.ds(base, EP)], typ_v.at[pl.ds(0, EP)])

        one = jnp.full((L,), 1.0, dtype=jnp.float32)
        zero = jnp.zeros((L,), dtype=jnp.float32)
        izero = jnp.zeros((L,), dtype=jnp.int32)

        def group(g, carry):
            for k in range(8):
                off = g * 128 + k * 16
                sv = src_v[pl.ds(off, L)]
                dv = dst_v[pl.ds(off, L)]
                tv = typ_v[pl.ds(off, L)]
                valid = (off + lane) < EP
                i1 = jnp.where(valid, sv * ROW + tv, izero)
                i2 = jnp.where(valid, dv * ROW + tv, izero)
                i3 = jnp.where(valid, HSTART + tv, izero)
                v1 = jnp.where(valid, one, zero)
                v2 = jnp.where(valid & (sv != dv), one, zero)
                idx_b[0, pl.ds(k * 16, L)] = i1
                idx_b[1, pl.ds(k * 16, L)] = i2
                idx_b[2, pl.ds(k * 16, L)] = i3
                val_b[0, pl.ds(k * 16, L)] = v1
                val_b[1, pl.ds(k * 16, L)] = v2
                val_b[2, pl.ds(k * 16, L)] = v1
            pltpu.sync_copy(val_b.at[0], table.at[idx_b.at[0]], add=True)
            pltpu.sync_copy(val_b.at[1], table.at[idx_b.at[1]], add=True)
            pltpu.sync_copy(val_b.at[2], table.at[idx_b.at[2]], add=True)
            return carry

        lax.fori_loop(0, NG, group, 0)
        plsc.subcore_barrier()

        pltpu.sync_copy(qent_hbm.at[pl.ds(s * QT, QT)], q_v)
        q = q_v[...]
        for m in range(QT):
            qm = lax.gather(
                q, jnp.full((L, 1), m, dtype=jnp.int32),
                lax.GatherDimensionNumbers(offset_dims=(),
                                           collapsed_slice_dims=(0,),
                                           start_index_map=(0,)),
                slice_sizes=(1,),
                mode=lax.GatherScatterMode.PROMISE_IN_BOUNDS)
            for sub in range(ROW // L):
                idx_g[m, pl.ds(sub * L, L)] = qm * ROW + sub * L + lane
        for m in range(QT):
            pltpu.sync_copy(table.at[idx_g.at[m]], gbuf.at[m])
        pltpu.sync_copy(gbuf, counts_out.at[c, pl.ds(s * QT, QT)])

        @pl.when(s == 0)
        def _():
            pltpu.sync_copy(table.at[pl.ds(HSTART, ROW)], hbuf)
            pltpu.sync_copy(hbuf, hist_out.at[c])

    return sc_fn


def _tc_kernel(B, R, D, E):
    def tc_fn(emb_ref, counts_ref, hist_ref, qrels_ref, dens_ref,
              w1a_ref, w1b_ref, w1c_ref, b1_ref, w2_ref, b2_ref,
              wg1_ref, bg1_ref, wg2_ref, bg2_ref, out_ref):
        counts_p = counts_ref[...]
        counts = counts_p[0] + counts_p[1]
        hist = hist_ref[0, :] + hist_ref[1, :]
        emb = emb_ref[...]
        qrels = qrels_ref[...]

        onehot = (qrels[:, None]
                  == lax.broadcasted_iota(jnp.int32, (B, R), 1)
                  ).astype(jnp.float32)
        countsR = counts[:, :R]

        qrel = jnp.sum(emb * onehot[:, :, None], axis=1)
        ent_sum = jnp.sum(emb * countsR[:, :, None], axis=1)
        deg = jnp.sum(counts, axis=1)
        ent_emb = jnp.where(deg[:, None] > 0,
                            ent_sum / jnp.maximum(deg, 1.0)[:, None], 0.0)

        rel_freq = jnp.sum(onehot * hist[None, :R], axis=1)
        inv_e = 1.0 / float(max(E, 1))
        s0 = jnp.minimum(rel_freq * inv_e, 1.0)
        s1 = jnp.minimum(deg * inv_e, 1.0)
        dens = dens_ref[0]

        dot = functools.partial(jnp.dot, precision=jax.lax.Precision.HIGHEST,
                                preferred_element_type=jnp.float32)
        h1 = dot(qrel, w1a_ref[...]) + dot(ent_emb, w1b_ref[...])
        w1c = w1c_ref[...]
        h1 = h1 + s0[:, None] * (w1c[0, :] + w1c[2, :])[None, :]
        h1 = h1 + s1[:, None] * w1c[1, :][None, :]
        h1 = h1 + dens * w1c[3, :][None, :]
        h1 = jax.nn.relu(h1 + b1_ref[...][None, :])
        h2 = jax.nn.relu(dot(h1, w2_ref[...]) + b2_ref[...][None, :])
        g3 = jax.nn.relu(dot(h2, wg1_ref[...]) + bg1_ref[...][None, :])
        z = jnp.sum(g3 * wg2_ref[...], axis=1) + bg2_ref[0]
        out_ref[...] = jax.nn.sigmoid(z)

    return tc_fn


def kernel(relation_embeddings, query_rels, query_entities, edge_index,
           edge_type, num_nodes, num_relations,
           W1, b1, W2, b2, Wg1, bg1, Wg2, bg2):
    B, R, D = relation_embeddings.shape
    E = edge_type.shape[0]

    src = edge_index[0].astype(jnp.int32)
    dst = edge_index[1].astype(jnp.int32)
    etype = edge_type.astype(jnp.int32)
    qent = query_entities.astype(jnp.int32)

    counts_raw, hist_raw = _sc_kernel(B, E)(src, dst, etype, qent)

    density = jnp.minimum(
        jnp.float32(E)
        / jnp.maximum(num_nodes * num_nodes, 1).astype(jnp.float32), 1.0)
    dens = jnp.reshape(density, (1,)).astype(jnp.float32)

    tc = pl.pallas_call(
        _tc_kernel(B, R, D, E),
        out_shape=jax.ShapeDtypeStruct((B,), jnp.float32),
        in_specs=[
            pl.BlockSpec(memory_space=pltpu.VMEM),
            pl.BlockSpec(memory_space=pltpu.VMEM),
            pl.BlockSpec(memory_space=pltpu.VMEM),
            pl.BlockSpec(memory_space=pltpu.VMEM),
            pl.BlockSpec(memory_space=pltpu.SMEM),
            pl.BlockSpec(memory_space=pltpu.VMEM),
            pl.BlockSpec(memory_space=pltpu.VMEM),
            pl.BlockSpec(memory_space=pltpu.VMEM),
            pl.BlockSpec(memory_space=pltpu.VMEM),
            pl.BlockSpec(memory_space=pltpu.VMEM),
            pl.BlockSpec(memory_space=pltpu.VMEM),
            pl.BlockSpec(memory_space=pltpu.VMEM),
            pl.BlockSpec(memory_space=pltpu.VMEM),
            pl.BlockSpec(memory_space=pltpu.VMEM),
            pl.BlockSpec(memory_space=pltpu.VMEM),
        ],
        out_specs=pl.BlockSpec(memory_space=pltpu.VMEM),
    )

    gate = tc(
        relation_embeddings.astype(jnp.float32),
        counts_raw, hist_raw,
        query_rels.astype(jnp.int32), dens,
        W1[:D, :], W1[D:2 * D, :], W1[2 * D:, :], b1,
        W2, b2, Wg1, bg1,
        jnp.reshape(Wg2, (1, -1)), bg2,
    )
    return gate


# speedup vs baseline: 14.5683x; 1.1752x over previous
import functools

import jax
import jax.numpy as jnp
from jax import lax
from jax.experimental import pallas as pl
from jax.experimental.pallas import tpu as pltpu
from jax.experimental.pallas import tpu_sc as plsc

N_NODES = 10000
NC, NS, L = 2, 16, 16

ROW = 128
HSTART = N_NODES * ROW
TBL = HSTART + NS * ROW
ZSLICE = TBL // NS
ZBUF = 8192
GB = 8


def _sc_kernel(B, E):
    EP = E // (NC * NS)
    EPP = ((EP + 127) // 128) * 128
    NG = EPP // 128
    QT = B // NS

    mesh = plsc.VectorSubcoreMesh(core_axis_name="c", subcore_axis_name="s",
                                  num_cores=NC, num_subcores=NS)

    @functools.partial(
        pl.kernel,
        out_type=(
            jax.ShapeDtypeStruct((NC, B, ROW), jnp.float32),
            jax.ShapeDtypeStruct((NC, NS, ROW), jnp.float32),
        ),
        mesh=mesh,
        scratch_types=[
            pltpu.VMEM_SHARED((TBL,), jnp.float32),
            pltpu.VMEM((EPP,), jnp.int32),
            pltpu.VMEM((EPP,), jnp.int32),
            pltpu.VMEM((EPP,), jnp.int32),
            pltpu.VMEM((GB, 3, 128), jnp.int32),
            pltpu.VMEM((GB, 3, 128), jnp.float32),
            pltpu.VMEM((L,), jnp.int32),
            pltpu.VMEM((QT, ROW), jnp.int32),
            pltpu.VMEM((QT, ROW), jnp.float32),
            pltpu.VMEM((ROW,), jnp.float32),
            pltpu.VMEM((ZBUF,), jnp.float32),
            pltpu.SemaphoreType.DMA,
            pltpu.SemaphoreType.DMA,
            pltpu.SemaphoreType.DMA,
        ],
    )
    def sc_fn(src_hbm, dst_hbm, typ_hbm, qent_hbm,
              counts_out, hist_out,
              table, src_v, dst_v, typ_v, idx_b, val_b, q_v, idx_g, gbuf,
              hbuf, zbuf, sem_e, sem_z, sem_s):
        c = lax.axis_index("c")
        s = lax.axis_index("s")
        wid = c * NS + s
        lane = jnp.arange(L, dtype=jnp.int32)

        base = wid * EP
        e_descs = [
            pltpu.async_copy(src_hbm.at[pl.ds(base, EP)],
                             src_v.at[pl.ds(0, EP)], sem_e),
            pltpu.async_copy(dst_hbm.at[pl.ds(base, EP)],
                             dst_v.at[pl.ds(0, EP)], sem_e),
            pltpu.async_copy(typ_hbm.at[pl.ds(base, EP)],
                             typ_v.at[pl.ds(0, EP)], sem_e),
        ]

        zvec = jnp.zeros((L,), dtype=jnp.float32)

        def zfill(j, carry):
            zbuf[pl.ds(j * L, L)] = zvec
            return carry

        lax.fori_loop(0, ZBUF // L, zfill, 0)
        z_descs = []
        off = 0
        while off < ZSLICE:
            n = min(ZBUF, ZSLICE - off)
            z_descs.append(pltpu.async_copy(
                zbuf.at[pl.ds(0, n)],
                table.at[pl.ds(s * ZSLICE + off, n)], sem_z))
            off += n
        for d in z_descs:
            d.wait()
        plsc.subcore_barrier()
        for d in e_descs:
            d.wait()

        one = jnp.full((L,), 1.0, dtype=jnp.float32)
        zero = jnp.zeros((L,), dtype=jnp.float32)
        izero = jnp.zeros((L,), dtype=jnp.int32)
        hbase = HSTART + s * ROW

        def batch(bi, carry):
            for j in range(GB):
                g = bi * GB + j
                for k in range(8):
                    off = g * 128 + k * 16
                    sv = src_v[pl.ds(off, L)]
                    dv = dst_v[pl.ds(off, L)]
                    tv = typ_v[pl.ds(off, L)]
                    valid = (off + lane) < EP
                    i1 = jnp.where(valid, sv * ROW + tv, izero)
                    i2 = jnp.where(valid, dv * ROW + tv, izero)
                    i3 = jnp.where(valid, hbase + tv, izero)
                    v1 = jnp.where(valid, one, zero)
                    v2 = jnp.where(valid & (sv != dv), one, zero)
                    idx_b[j, 0, pl.ds(k * 16, L)] = i1
                    idx_b[j, 1, pl.ds(k * 16, L)] = i2
                    idx_b[j, 2, pl.ds(k * 16, L)] = i3
                    val_b[j, 0, pl.ds(k * 16, L)] = v1
                    val_b[j, 1, pl.ds(k * 16, L)] = v2
                    val_b[j, 2, pl.ds(k * 16, L)] = v1
            descs = []
            for j in range(GB):
                for r in range(3):
                    descs.append(pltpu.async_copy(
                        val_b.at[j, r], table.at[idx_b.at[j, r]], sem_s,
                        add=True))
            for d in descs:
                d.wait()
            return carry

        lax.fori_loop(0, NG // GB, batch, 0)
        plsc.subcore_barrier()

        pltpu.sync_copy(qent_hbm.at[pl.ds(s * QT, QT)], q_v)
        q = q_v[...]
        for m in range(QT):
            qm = lax.gather(
                q, jnp.full((L, 1), m, dtype=jnp.int32),
                lax.GatherDimensionNumbers(offset_dims=(),
                                           collapsed_slice_dims=(0,),
                                           start_index_map=(0,)),
                slice_sizes=(1,),
                mode=lax.GatherScatterMode.PROMISE_IN_BOUNDS)
            for sub in range(ROW // L):
                idx_g[m, pl.ds(sub * L, L)] = qm * ROW + sub * L + lane
        g_descs = [pltpu.async_copy(table.at[idx_g.at[m]], gbuf.at[m], sem_z)
                   for m in range(QT)]
        for d in g_descs:
            d.wait()
        pltpu.sync_copy(gbuf, counts_out.at[c, pl.ds(s * QT, QT)])

        pltpu.sync_copy(table.at[pl.ds(hbase, ROW)], hbuf)
        pltpu.sync_copy(hbuf, hist_out.at[c, s])

    return sc_fn


def _tc_kernel(B, R, D, E):
    def tc_fn(emb_ref, counts_ref, hist_ref, qrels_ref, dens_ref,
              w1a_ref, w1b_ref, w1c_ref, b1_ref, w2_ref, b2_ref,
              wg1_ref, bg1_ref, wg2_ref, bg2_ref, out_ref):
        counts_p = counts_ref[...]
        counts = counts_p[0] + counts_p[1]
        hist = jnp.sum(jnp.reshape(hist_ref[...], (NC * NS, ROW)), axis=0)
        emb = emb_ref[...]
        qrels = qrels_ref[...]

        onehot = (qrels[:, None]
                  == lax.broadcasted_iota(jnp.int32, (B, R), 1)
                  ).astype(jnp.float32)
        countsR = counts[:, :R]

        qrel = jnp.sum(emb * onehot[:, :, None], axis=1)
        ent_sum = jnp.sum(emb * countsR[:, :, None], axis=1)
        deg = jnp.sum(counts, axis=1)
        ent_emb = jnp.where(deg[:, None] > 0,
                            ent_sum / jnp.maximum(deg, 1.0)[:, None], 0.0)

        rel_freq = jnp.sum(onehot * hist[None, :R], axis=1)
        inv_e = 1.0 / float(max(E, 1))
        s0 = jnp.minimum(rel_freq * inv_e, 1.0)
        s1 = jnp.minimum(deg * inv_e, 1.0)
        dens = dens_ref[0]

        dot = functools.partial(jnp.dot, precision=jax.lax.Precision.HIGHEST,
                                preferred_element_type=jnp.float32)
        h1 = dot(qrel, w1a_ref[...]) + dot(ent_emb, w1b_ref[...])
        w1c = w1c_ref[...]
        h1 = h1 + s0[:, None] * (w1c[0, :] + w1c[2, :])[None, :]
        h1 = h1 + s1[:, None] * w1c[1, :][None, :]
        h1 = h1 + dens * w1c[3, :][None, :]
        h1 = jax.nn.relu(h1 + b1_ref[...][None, :])
        h2 = jax.nn.relu(dot(h1, w2_ref[...]) + b2_ref[...][None, :])
        g3 = jax.nn.relu(dot(h2, wg1_ref[...]) + bg1_ref[...][None, :])
        z = jnp.sum(g3 * wg2_ref[...], axis=1) + bg2_ref[0]
        out_ref[...] = jax.nn.sigmoid(z)

    return tc_fn


def kernel(relation_embeddings, query_rels, query_entities, edge_index,
           edge_type, num_nodes, num_relations,
           W1, b1, W2, b2, Wg1, bg1, Wg2, bg2):
    B, R, D = relation_embeddings.shape
    E = edge_type.shape[0]

    src = edge_index[0].astype(jnp.int32)
    dst = edge_index[1].astype(jnp.int32)
    etype = edge_type.astype(jnp.int32)
    qent = query_entities.astype(jnp.int32)

    counts_raw, hist_raw = _sc_kernel(B, E)(src, dst, etype, qent)

    density = jnp.minimum(
        jnp.float32(E)
        / jnp.maximum(num_nodes * num_nodes, 1).astype(jnp.float32), 1.0)
    dens = jnp.reshape(density, (1,)).astype(jnp.float32)

    tc = pl.pallas_call(
        _tc_kernel(B, R, D, E),
        out_shape=jax.ShapeDtypeStruct((B,), jnp.float32),
        in_specs=[
            pl.BlockSpec(memory_space=pltpu.VMEM),
            pl.BlockSpec(memory_space=pltpu.VMEM),
            pl.BlockSpec(memory_space=pltpu.VMEM),
            pl.BlockSpec(memory_space=pltpu.VMEM),
            pl.BlockSpec(memory_space=pltpu.SMEM),
            pl.BlockSpec(memory_space=pltpu.VMEM),
            pl.BlockSpec(memory_space=pltpu.VMEM),
            pl.BlockSpec(memory_space=pltpu.VMEM),
            pl.BlockSpec(memory_space=pltpu.VMEM),
            pl.BlockSpec(memory_space=pltpu.VMEM),
            pl.BlockSpec(memory_space=pltpu.VMEM),
            pl.BlockSpec(memory_space=pltpu.VMEM),
            pl.BlockSpec(memory_space=pltpu.VMEM),
            pl.BlockSpec(memory_space=pltpu.VMEM),
            pl.BlockSpec(memory_space=pltpu.VMEM),
        ],
        out_specs=pl.BlockSpec(memory_space=pltpu.VMEM),
    )

    gate = tc(
        relation_embeddings.astype(jnp.float32),
        counts_raw, hist_raw,
        query_rels.astype(jnp.int32), dens,
        W1[:D, :], W1[D:2 * D, :], W1[2 * D:, :], b1,
        W2, b2, Wg1, bg1,
        jnp.reshape(Wg2, (1, -1)), bg2,
    )
    return gate


# speedup vs baseline: 14.5877x; 1.0013x over previous
import functools

import jax
import jax.numpy as jnp
from jax import lax
from jax.experimental import pallas as pl
from jax.experimental.pallas import tpu as pltpu
from jax.experimental.pallas import tpu_sc as plsc

N_NODES = 10000
NC, NS, L = 2, 16, 16

ROW = 128
HSTART = N_NODES * ROW
TBL = HSTART + NS * ROW
ZSLICE = TBL // NS
ZBUF = 8192
GB = 8


def _sc_kernel(B, E):
    EP = E // (NC * NS)
    EPP = ((EP + 127) // 128) * 128
    NG = EPP // 128
    QT = B // NS

    mesh = plsc.VectorSubcoreMesh(core_axis_name="c", subcore_axis_name="s",
                                  num_cores=NC, num_subcores=NS)

    @functools.partial(
        pl.kernel,
        out_type=(
            jax.ShapeDtypeStruct((NC, B, ROW), jnp.float32),
            jax.ShapeDtypeStruct((NC, NS, ROW), jnp.float32),
        ),
        mesh=mesh,
        scratch_types=[
            pltpu.VMEM_SHARED((TBL,), jnp.float32),
            pltpu.VMEM((EPP,), jnp.int32),
            pltpu.VMEM((EPP,), jnp.int32),
            pltpu.VMEM((EPP,), jnp.int32),
            pltpu.VMEM((GB, 3, 128), jnp.int32),
            pltpu.VMEM((GB, 3, 128), jnp.float32),
            pltpu.VMEM((L,), jnp.int32),
            pltpu.VMEM((QT, ROW), jnp.int32),
            pltpu.VMEM((QT, ROW), jnp.float32),
            pltpu.VMEM((ROW,), jnp.float32),
            pltpu.VMEM((ZBUF,), jnp.float32),
            pltpu.SemaphoreType.DMA,
            pltpu.SemaphoreType.DMA,
            pltpu.SemaphoreType.DMA,
        ],
    )
    def sc_fn(src_hbm, dst_hbm, typ_hbm, qent_hbm,
              counts_out, hist_out,
              table, src_v, dst_v, typ_v, idx_b, val_b, q_v, idx_g,
              gbuf, hbuf, zbuf, sem_e, sem_z, sem_s):
        c = lax.axis_index("c")
        s = lax.axis_index("s")
        wid = c * NS + s
        lane = jnp.arange(L, dtype=jnp.int32)

        base = wid * EP
        e_descs = [
            pltpu.async_copy(src_hbm.at[pl.ds(base, EP)],
                             src_v.at[pl.ds(0, EP)], sem_e),
            pltpu.async_copy(dst_hbm.at[pl.ds(base, EP)],
                             dst_v.at[pl.ds(0, EP)], sem_e),
            pltpu.async_copy(typ_hbm.at[pl.ds(base, EP)],
                             typ_v.at[pl.ds(0, EP)], sem_e),
        ]

        zvec = jnp.zeros((L,), dtype=jnp.float32)

        def zfill(j, carry):
            zbuf[pl.ds(j * L, L)] = zvec
            return carry

        lax.fori_loop(0, ZBUF // L, zfill, 0)
        z_descs = []
        off = 0
        while off < ZSLICE:
            n = min(ZBUF, ZSLICE - off)
            z_descs.append(pltpu.async_copy(
                zbuf.at[pl.ds(0, n)],
                table.at[pl.ds(s * ZSLICE + off, n)], sem_z))
            off += n
        for d in z_descs:
            d.wait()
        plsc.subcore_barrier()
        for d in e_descs:
            d.wait()

        one = jnp.full((L,), 1.0, dtype=jnp.float32)
        zero = jnp.zeros((L,), dtype=jnp.float32)
        izero = jnp.zeros((L,), dtype=jnp.int32)
        hbase = HSTART + s * ROW

        def emit_group(gbase, j):
            for k in range(8):
                off = gbase + k * 16
                sv = src_v[pl.ds(off, L)]
                dv = dst_v[pl.ds(off, L)]
                tv = typ_v[pl.ds(off, L)]
                valid = (off + lane) < EP
                i1 = jnp.where(valid, sv * ROW + tv, izero)
                i2 = jnp.where(valid, dv * ROW + tv, izero)
                i3 = jnp.where(valid, hbase + tv, izero)
                v1 = jnp.where(valid, one, zero)
                v2 = jnp.where(valid & (sv != dv), one, zero)
                idx_b[j, 0, pl.ds(k * 16, L)] = i1
                idx_b[j, 1, pl.ds(k * 16, L)] = i2
                idx_b[j, 2, pl.ds(k * 16, L)] = i3
                val_b[j, 0, pl.ds(k * 16, L)] = v1
                val_b[j, 1, pl.ds(k * 16, L)] = v2
                val_b[j, 2, pl.ds(k * 16, L)] = v1
            for r in range(3):
                pltpu.async_copy(val_b.at[j, r], table.at[idx_b.at[j, r]],
                                 sem_s, add=True)

        def drain_slot(j):
            for r in range(3):
                pltpu.make_async_copy(val_b.at[j, r],
                                      table.at[idx_b.at[j, r]], sem_s).wait()

        for j in range(GB):
            emit_group(j * 128, j)

        def ring(bi, carry):
            for j in range(GB):
                drain_slot(j)
                emit_group((bi * GB + j) * 128, j)
            return carry

        lax.fori_loop(1, NG // GB, ring, 0)
        for j in range(GB):
            drain_slot(j)

        pltpu.sync_copy(table.at[pl.ds(hbase, ROW)], hbuf)
        h_desc = pltpu.async_copy(hbuf, hist_out.at[c, s], sem_e)
        plsc.subcore_barrier()

        pltpu.sync_copy(qent_hbm.at[pl.ds(s * QT, QT)], q_v)
        q = q_v[...]
        for m in range(QT):
            qm = lax.gather(
                q, jnp.full((L, 1), m, dtype=jnp.int32),
                lax.GatherDimensionNumbers(offset_dims=(),
                                           collapsed_slice_dims=(0,),
                                           start_index_map=(0,)),
                slice_sizes=(1,),
                mode=lax.GatherScatterMode.PROMISE_IN_BOUNDS)
            for sub in range(ROW // L):
                idx_g[m, pl.ds(sub * L, L)] = qm * ROW + sub * L + lane
        g_descs = [pltpu.async_copy(table.at[idx_g.at[m]], gbuf.at[m], sem_z)
                   for m in range(QT)]
        for d in g_descs:
            d.wait()
        pltpu.sync_copy(gbuf, counts_out.at[c, pl.ds(s * QT, QT)])
        h_desc.wait()

    return sc_fn


def _tc_kernel(B, R, D, E):
    def tc_fn(emb_ref, counts_ref, hist_ref, qrels_ref, dens_ref,
              w1a_ref, w1b_ref, w1c_ref, b1_ref, w2_ref, b2_ref,
              wg1_ref, bg1_ref, wg2_ref, bg2_ref, out_ref):
        counts_p = counts_ref[...]
        counts = counts_p[0] + counts_p[1]
        hist = jnp.sum(jnp.reshape(hist_ref[...], (NC * NS, ROW)), axis=0)
        emb = emb_ref[...]
        qrels = qrels_ref[...]

        onehot = (qrels[:, None]
                  == lax.broadcasted_iota(jnp.int32, (B, R), 1)
                  ).astype(jnp.float32)
        countsR = counts[:, :R]

        qrel = jnp.sum(emb * onehot[:, :, None], axis=1)
        ent_sum = jnp.sum(emb * countsR[:, :, None], axis=1)
        deg = jnp.sum(counts, axis=1)
        ent_emb = jnp.where(deg[:, None] > 0,
                            ent_sum / jnp.maximum(deg, 1.0)[:, None], 0.0)

        rel_freq = jnp.sum(onehot * hist[None, :R], axis=1)
        inv_e = 1.0 / float(max(E, 1))
        s0 = jnp.minimum(rel_freq * inv_e, 1.0)
        s1 = jnp.minimum(deg * inv_e, 1.0)
        dens = dens_ref[0]

        dot = functools.partial(jnp.dot, precision=jax.lax.Precision.HIGHEST,
                                preferred_element_type=jnp.float32)
        h1 = dot(qrel, w1a_ref[...]) + dot(ent_emb, w1b_ref[...])
        w1c = w1c_ref[...]
        h1 = h1 + s0[:, None] * (w1c[0, :] + w1c[2, :])[None, :]
        h1 = h1 + s1[:, None] * w1c[1, :][None, :]
        h1 = h1 + dens * w1c[3, :][None, :]
        h1 = jax.nn.relu(h1 + b1_ref[...][None, :])
        h2 = jax.nn.relu(dot(h1, w2_ref[...]) + b2_ref[...][None, :])
        g3 = jax.nn.relu(dot(h2, wg1_ref[...]) + bg1_ref[...][None, :])
        z = jnp.sum(g3 * wg2_ref[...], axis=1) + bg2_ref[0]
        out_ref[...] = jax.nn.sigmoid(z)

    return tc_fn


def kernel(relation_embeddings, query_rels, query_entities, edge_index,
           edge_type, num_nodes, num_relations,
           W1, b1, W2, b2, Wg1, bg1, Wg2, bg2):
    B, R, D = relation_embeddings.shape
    E = edge_type.shape[0]

    src = edge_index[0].astype(jnp.int32)
    dst = edge_index[1].astype(jnp.int32)
    etype = edge_type.astype(jnp.int32)
    qent = query_entities.astype(jnp.int32)

    counts_raw, hist_raw = _sc_kernel(B, E)(src, dst, etype, qent)

    density = jnp.minimum(
        jnp.float32(E)
        / jnp.maximum(num_nodes * num_nodes, 1).astype(jnp.float32), 1.0)
    dens = jnp.reshape(density, (1,)).astype(jnp.float32)

    tc = pl.pallas_call(
        _tc_kernel(B, R, D, E),
        out_shape=jax.ShapeDtypeStruct((B,), jnp.float32),
        in_specs=[
            pl.BlockSpec(memory_space=pltpu.VMEM),
            pl.BlockSpec(memory_space=pltpu.VMEM),
            pl.BlockSpec(memory_space=pltpu.VMEM),
            pl.BlockSpec(memory_space=pltpu.VMEM),
            pl.BlockSpec(memory_space=pltpu.SMEM),
            pl.BlockSpec(memory_space=pltpu.VMEM),
            pl.BlockSpec(memory_space=pltpu.VMEM),
            pl.BlockSpec(memory_space=pltpu.VMEM),
            pl.BlockSpec(memory_space=pltpu.VMEM),
            pl.BlockSpec(memory_space=pltpu.VMEM),
            pl.BlockSpec(memory_space=pltpu.VMEM),
            pl.BlockSpec(memory_space=pltpu.VMEM),
            pl.BlockSpec(memory_space=pltpu.VMEM),
            pl.BlockSpec(memory_space=pltpu.VMEM),
            pl.BlockSpec(memory_space=pltpu.VMEM),
        ],
        out_specs=pl.BlockSpec(memory_space=pltpu.VMEM),
    )

    gate = tc(
        relation_embeddings.astype(jnp.float32),
        counts_raw, hist_raw,
        query_rels.astype(jnp.int32), dens,
        W1[:D, :], W1[D:2 * D, :], W1[2 * D:, :], b1,
        W2, b2, Wg1, bg1,
        jnp.reshape(Wg2, (1, -1)), bg2,
    )
    return gate


# speedup vs baseline: 14.6040x; 1.0011x over previous
import functools

import jax
import jax.numpy as jnp
from jax import lax
from jax.experimental import pallas as pl
from jax.experimental.pallas import tpu as pltpu
from jax.experimental.pallas import tpu_sc as plsc

N_NODES = 10000
NC, NS, L = 2, 16, 16

ROW = 128
HSTART = N_NODES * ROW
TBL = HSTART + NS * ROW
ZSLICE = TBL // NS
ZBUF = 8192
GB = 8


def _sc_kernel(B, E):
    EP = E // (NC * NS)
    EPP = ((EP + 127) // 128) * 128
    NG = EPP // 128
    QT = B // NS

    mesh = plsc.VectorSubcoreMesh(core_axis_name="c", subcore_axis_name="s",
                                  num_cores=NC, num_subcores=NS)

    @functools.partial(
        pl.kernel,
        out_type=(
            jax.ShapeDtypeStruct((NC, B, ROW), jnp.float32),
            jax.ShapeDtypeStruct((NC, NS, ROW), jnp.float32),
        ),
        mesh=mesh,
        scratch_types=[
            pltpu.VMEM_SHARED((TBL,), jnp.float32),
            pltpu.VMEM((EPP,), jnp.int32),
            pltpu.VMEM((EPP,), jnp.int32),
            pltpu.VMEM((EPP,), jnp.int32),
            pltpu.VMEM((GB, 3, 128), jnp.int32),
            pltpu.VMEM((GB, 3, 128), jnp.float32),
            pltpu.VMEM((L,), jnp.int32),
            pltpu.VMEM((QT, ROW), jnp.int32),
            pltpu.VMEM((QT, ROW), jnp.float32),
            pltpu.VMEM((ROW,), jnp.float32),
            pltpu.VMEM((ZBUF,), jnp.float32),
            pltpu.SemaphoreType.DMA,
            pltpu.SemaphoreType.DMA,
            pltpu.SemaphoreType.DMA,
        ],
    )
    def sc_fn(src_hbm, dst_hbm, typ_hbm, qent_hbm,
              counts_out, hist_out,
              table, src_v, dst_v, typ_v, idx_b, val_b, q_v, idx_g,
              gbuf, hbuf, zbuf, sem_e, sem_z, sem_s):
        c = lax.axis_index("c")
        s = lax.axis_index("s")
        wid = c * NS + s
        lane = jnp.arange(L, dtype=jnp.int32)

        base = wid * EP
        e_descs = [
            pltpu.async_copy(src_hbm.at[pl.ds(base, EP)],
                             src_v.at[pl.ds(0, EP)], sem_e),
            pltpu.async_copy(dst_hbm.at[pl.ds(base, EP)],
                             dst_v.at[pl.ds(0, EP)], sem_e),
            pltpu.async_copy(typ_hbm.at[pl.ds(base, EP)],
                             typ_v.at[pl.ds(0, EP)], sem_e),
        ]

        zvec = jnp.zeros((L,), dtype=jnp.float32)

        def zfill(j, carry):
            zbuf[pl.ds(j * L, L)] = zvec
            return carry

        with jax.named_scope("p0_zero"):
            lax.fori_loop(0, ZBUF // L, zfill, 0)
            z_descs = []
            off = 0
            while off < ZSLICE:
                n = min(ZBUF, ZSLICE - off)
                z_descs.append(pltpu.async_copy(
                    zbuf.at[pl.ds(0, n)],
                    table.at[pl.ds(s * ZSLICE + off, n)], sem_z))
                off += n
            for d in z_descs:
                d.wait()
            plsc.subcore_barrier()
            for d in e_descs:
                d.wait()

        one = jnp.full((L,), 1.0, dtype=jnp.float32)
        zero = jnp.zeros((L,), dtype=jnp.float32)
        izero = jnp.zeros((L,), dtype=jnp.int32)
        hbase = HSTART + s * ROW

        def emit_group(gbase, j):
            for k in range(8):
                off = gbase + k * 16
                sv = src_v[pl.ds(off, L)]
                dv = dst_v[pl.ds(off, L)]
                tv = typ_v[pl.ds(off, L)]
                valid = (off + lane) < EP
                i1 = jnp.where(valid, sv * ROW + tv, izero)
                i2 = jnp.where(valid, dv * ROW + tv, izero)
                i3 = jnp.where(valid, hbase + tv, izero)
                v1 = jnp.where(valid, one, zero)
                v2 = jnp.where(valid & (sv != dv), one, zero)
                idx_b[j, 0, pl.ds(k * 16, L)] = i1
                idx_b[j, 1, pl.ds(k * 16, L)] = i2
                idx_b[j, 2, pl.ds(k * 16, L)] = i3
                val_b[j, 0, pl.ds(k * 16, L)] = v1
                val_b[j, 1, pl.ds(k * 16, L)] = v2
                val_b[j, 2, pl.ds(k * 16, L)] = v1
            for r in range(3):
                pltpu.async_copy(val_b.at[j, r], table.at[idx_b.at[j, r]],
                                 sem_s, add=True)

        def drain_slot(j):
            for r in range(3):
                pltpu.make_async_copy(val_b.at[j, r],
                                      table.at[idx_b.at[j, r]], sem_s).wait()

        with jax.named_scope("p1_scatter"):
            for j in range(GB):
                emit_group(j * 128, j)

            def ring(bi, carry):
                for j in range(GB):
                    drain_slot(j)
                    emit_group((bi * GB + j) * 128, j)
                return carry

            lax.fori_loop(1, NG // GB, ring, 0)
            for j in range(GB):
                drain_slot(j)

        with jax.named_scope("p2_gather"):
            pltpu.sync_copy(table.at[pl.ds(hbase, ROW)], hbuf)
            h_desc = pltpu.async_copy(hbuf, hist_out.at[c, s], sem_e)
            plsc.subcore_barrier()

            pltpu.sync_copy(qent_hbm.at[pl.ds(s * QT, QT)], q_v)
            q = q_v[...]
            for m in range(QT):
                qm = lax.gather(
                    q, jnp.full((L, 1), m, dtype=jnp.int32),
                    lax.GatherDimensionNumbers(offset_dims=(),
                                               collapsed_slice_dims=(0,),
                                               start_index_map=(0,)),
                    slice_sizes=(1,),
                    mode=lax.GatherScatterMode.PROMISE_IN_BOUNDS)
                for sub in range(ROW // L):
                    idx_g[m, pl.ds(sub * L, L)] = qm * ROW + sub * L + lane
            g_descs = [pltpu.async_copy(table.at[idx_g.at[m]], gbuf.at[m],
                                        sem_z)
                       for m in range(QT)]
            for d in g_descs:
                d.wait()
            pltpu.sync_copy(gbuf, counts_out.at[c, pl.ds(s * QT, QT)])
            h_desc.wait()

    return sc_fn


def _tc_kernel(B, R, D, E):
    def tc_fn(emb_ref, counts_ref, hist_ref, qrels_ref, dens_ref,
              w1a_ref, w1b_ref, w1c_ref, b1_ref, w2_ref, b2_ref,
              wg1_ref, bg1_ref, wg2_ref, bg2_ref, out_ref):
        counts_p = counts_ref[...]
        counts = counts_p[0] + counts_p[1]
        hist = jnp.sum(jnp.reshape(hist_ref[...], (NC * NS, ROW)), axis=0)
        emb = emb_ref[...]
        qrels = qrels_ref[...]

        onehot = (qrels[:, None]
                  == lax.broadcasted_iota(jnp.int32, (B, R), 1)
                  ).astype(jnp.float32)
        countsR = counts[:, :R]

        qrel = jnp.sum(emb * onehot[:, :, None], axis=1)
        ent_sum = jnp.sum(emb * countsR[:, :, None], axis=1)
        deg = jnp.sum(counts, axis=1)
        ent_emb = jnp.where(deg[:, None] > 0,
                            ent_sum / jnp.maximum(deg, 1.0)[:, None], 0.0)

        rel_freq = jnp.sum(onehot * hist[None, :R], axis=1)
        inv_e = 1.0 / float(max(E, 1))
        s0 = jnp.minimum(rel_freq * inv_e, 1.0)
        s1 = jnp.minimum(deg * inv_e, 1.0)
        dens = dens_ref[0]

        dot = functools.partial(jnp.dot, precision=jax.lax.Precision.HIGHEST,
                                preferred_element_type=jnp.float32)
        h1 = dot(qrel, w1a_ref[...]) + dot(ent_emb, w1b_ref[...])
        w1c = w1c_ref[...]
        h1 = h1 + s0[:, None] * (w1c[0, :] + w1c[2, :])[None, :]
        h1 = h1 + s1[:, None] * w1c[1, :][None, :]
        h1 = h1 + dens * w1c[3, :][None, :]
        h1 = jax.nn.relu(h1 + b1_ref[...][None, :])
        h2 = jax.nn.relu(dot(h1, w2_ref[...]) + b2_ref[...][None, :])
        g3 = jax.nn.relu(dot(h2, wg1_ref[...]) + bg1_ref[...][None, :])
        z = jnp.sum(g3 * wg2_ref[...], axis=1) + bg2_ref[0]
        out_ref[...] = jax.nn.sigmoid(z)

    return tc_fn


def kernel(relation_embeddings, query_rels, query_entities, edge_index,
           edge_type, num_nodes, num_relations,
           W1, b1, W2, b2, Wg1, bg1, Wg2, bg2):
    B, R, D = relation_embeddings.shape
    E = edge_type.shape[0]

    src = edge_index[0].astype(jnp.int32)
    dst = edge_index[1].astype(jnp.int32)
    etype = edge_type.astype(jnp.int32)
    qent = query_entities.astype(jnp.int32)

    counts_raw, hist_raw = _sc_kernel(B, E)(src, dst, etype, qent)

    density = jnp.minimum(
        jnp.float32(E)
        / jnp.maximum(num_nodes * num_nodes, 1).astype(jnp.float32), 1.0)
    dens = jnp.reshape(density, (1,)).astype(jnp.float32)

    tc = pl.pallas_call(
        _tc_kernel(B, R, D, E),
        out_shape=jax.ShapeDtypeStruct((B,), jnp.float32),
        in_specs=[
            pl.BlockSpec(memory_space=pltpu.VMEM),
            pl.BlockSpec(memory_space=pltpu.VMEM),
            pl.BlockSpec(memory_space=pltpu.VMEM),
            pl.BlockSpec(memory_space=pltpu.VMEM),
            pl.BlockSpec(memory_space=pltpu.SMEM),
            pl.BlockSpec(memory_space=pltpu.VMEM),
            pl.BlockSpec(memory_space=pltpu.VMEM),
            pl.BlockSpec(memory_space=pltpu.VMEM),
            pl.BlockSpec(memory_space=pltpu.VMEM),
            pl.BlockSpec(memory_space=pltpu.VMEM),
            pl.BlockSpec(memory_space=pltpu.VMEM),
            pl.BlockSpec(memory_space=pltpu.VMEM),
            pl.BlockSpec(memory_space=pltpu.VMEM),
            pl.BlockSpec(memory_space=pltpu.VMEM),
            pl.BlockSpec(memory_space=pltpu.VMEM),
        ],
        out_specs=pl.BlockSpec(memory_space=pltpu.VMEM),
    )

    gate = tc(
        relation_embeddings.astype(jnp.float32),
        counts_raw, hist_raw,
        query_rels.astype(jnp.int32), dens,
        W1[:D, :], W1[D:2 * D, :], W1[2 * D:, :], b1,
        W2, b2, Wg1, bg1,
        jnp.reshape(Wg2, (1, -1)), bg2,
    )
    return gate
